# baseline (speedup 1.0000x reference)
import functools

import jax
import jax.numpy as jnp
from jax import lax
from jax.experimental import pallas as pl
from jax.experimental.pallas import tpu as pltpu
from jax.experimental.pallas import tpu_sc as plsc

N = 10000
E = 320000
D = 128
NC = 2
NS = 16
NW = NC * NS
E_PER_SC = E // NC
E_PER_TILE = E_PER_SC // NS
CH = 112
N_CHUNKS = E_PER_TILE // CH
E_TAIL = E_PER_TILE - N_CHUNKS * CH
ROWS_PER_TILE = 632
N_ROW_PAD = NS * ROWS_PER_TILE
N_PAD = 10240
ROWS1D_PER_TILE = N_PAD // NS

_mesh = plsc.VectorSubcoreMesh(core_axis_name="c", subcore_axis_name="s")


NBUF = 2
N_MAIN = (N_CHUNKS // NBUF) * NBUF
N_GROUPS = N_MAIN // NBUF
ZCOPY_TILES = 10
ZROWS = N // ZCOPY_TILES


@functools.partial(
    pl.kernel,
    mesh=_mesh,
    out_type=jax.ShapeDtypeStruct((NC, N, D), jnp.float32),
    scratch_types=[
        pltpu.VMEM((E_PER_TILE,), jnp.int32),
        pltpu.VMEM((E_PER_TILE,), jnp.int32),
        [pltpu.VMEM((CH, D), jnp.float32) for _ in range(NBUF)],
        [pltpu.SemaphoreType.DMA for _ in range(NBUF)],
        [pltpu.SemaphoreType.DMA for _ in range(NBUF)],
        pltpu.SemaphoreType.DMA,
        pltpu.SemaphoreType.DMA,
        pltpu.VMEM_SHARED((N, D), jnp.float32),
    ],
)
def _seg_sum_rows(h_hbm, src_hbm, dst_hbm, zeros_hbm, parts_hbm,
                  src_v, dst_v, rows_v, gsems, ssems, isem0, isem1, accum_sh):
    c = lax.axis_index("c")
    s = lax.axis_index("s")
    ebase = c * E_PER_SC + s * E_PER_TILE
    pltpu.async_copy(src_hbm.at[pl.ds(ebase, E_PER_TILE)], src_v, isem0)
    pltpu.async_copy(dst_hbm.at[pl.ds(ebase, E_PER_TILE)], dst_v, isem1)

    @pl.when(s < ZCOPY_TILES)
    def _():
        pltpu.sync_copy(zeros_hbm, accum_sh.at[pl.ds(s * ZROWS, ZROWS)])

    pltpu.make_async_copy(src_hbm.at[pl.ds(ebase, E_PER_TILE)], src_v,
                          isem0).wait()
    pltpu.make_async_copy(dst_hbm.at[pl.ds(ebase, E_PER_TILE)], dst_v,
                          isem1).wait()
    plsc.subcore_barrier()

    def sidx(chunk):
        return src_v.at[pl.ds(chunk * CH, CH)]

    def didx(chunk):
        return dst_v.at[pl.ds(chunk * CH, CH)]

    def fire(chunk, b):
        pltpu.async_copy(h_hbm.at[sidx(chunk)], rows_v[b], gsems[b])

    def wait_gather_start_scatter(chunk, b):
        pltpu.make_async_copy(h_hbm.at[sidx(chunk)], rows_v[b],
                              gsems[b]).wait()
        pltpu.async_copy(rows_v[b], accum_sh.at[didx(chunk)], ssems[b],
                         add=True)

    def wait_scatter(chunk, b):
        pltpu.make_async_copy(rows_v[b], accum_sh.at[didx(chunk)],
                              ssems[b]).wait()

    for b in range(NBUF):
        fire(b, b)

    def body(g, carry):
        for b in range(NBUF):
            wait_gather_start_scatter(g * NBUF + b, b)
        for b in range(NBUF):
            wait_scatter(g * NBUF + b, b)
            fire((g + 1) * NBUF + b, b)
        return carry

    lax.fori_loop(0, N_GROUPS - 1, body, 0)
    g_last = N_GROUPS - 1
    for b in range(NBUF):
        wait_gather_start_scatter(g_last * NBUF + b, b)
    for b in range(NBUF):
        wait_scatter(g_last * NBUF + b, b)
    for t in range(N_MAIN, N_CHUNKS):
        fire(t, 0)
        wait_gather_start_scatter(t, 0)
        wait_scatter(t, 0)

    if E_TAIL:
        ti = src_v.at[pl.ds(N_CHUNKS * CH, E_TAIL)]
        to = dst_v.at[pl.ds(N_CHUNKS * CH, E_TAIL)]
        tr = rows_v[0].at[pl.ds(0, E_TAIL)]
        pltpu.async_copy(h_hbm.at[ti], tr, gsems[0])
        pltpu.make_async_copy(h_hbm.at[ti], tr, gsems[0]).wait()
        pltpu.sync_copy(tr, accum_sh.at[to], add=True)

    plsc.subcore_barrier()

    @pl.when(s < ZCOPY_TILES)
    def _():
        pltpu.sync_copy(accum_sh.at[pl.ds(s * ZROWS, ZROWS)],
                        parts_hbm.at[c, pl.ds(s * ZROWS, ZROWS)])


@functools.partial(
    pl.kernel,
    mesh=_mesh,
    out_type=jax.ShapeDtypeStruct((NC, N_PAD), jnp.float32),
    scratch_types=[
        [pltpu.VMEM((CH,), jnp.int32) for _ in range(NBUF)],
        [pltpu.VMEM((CH,), jnp.float32) for _ in range(NBUF)],
        [pltpu.SemaphoreType.DMA for _ in range(NBUF)],
        pltpu.VMEM_SHARED((N_PAD,), jnp.float32),
    ],
)
def _seg_sum_edge_scalar(dst_hbm, ef_hbm, zeros1d_hbm, parts_hbm,
                         dst_v, ef_v, sems, accum_sh):
    c = lax.axis_index("c")
    s = lax.axis_index("s")
    row0 = s * ROWS1D_PER_TILE
    pltpu.sync_copy(zeros1d_hbm, accum_sh.at[pl.ds(row0, ROWS1D_PER_TILE)])
    plsc.subcore_barrier()

    ebase = c * E_PER_SC + s * E_PER_TILE

    def body(i, carry):
        for b in range(NBUF):
            base = ebase + (i * NBUF + b) * CH
            pltpu.sync_copy(dst_hbm.at[pl.ds(base, CH)], dst_v[b])
            pltpu.sync_copy(ef_hbm.at[pl.ds(base, CH)], ef_v[b])
            pltpu.async_copy(ef_v[b], accum_sh.at[dst_v[b]], sems[b],
                             add=True)
        for b in range(NBUF):
            pltpu.make_async_copy(ef_v[b], accum_sh.at[dst_v[b]],
                                  sems[b]).wait()
        return carry

    lax.fori_loop(0, N_CHUNKS // NBUF, body, 0)
    for t in range(N_MAIN, N_CHUNKS):
        base = ebase + t * CH
        pltpu.sync_copy(dst_hbm.at[pl.ds(base, CH)], dst_v[0])
        pltpu.sync_copy(ef_hbm.at[pl.ds(base, CH)], ef_v[0])
        pltpu.sync_copy(ef_v[0], accum_sh.at[dst_v[0]], add=True)
    if E_TAIL:
        base = ebase + N_CHUNKS * CH
        ti = dst_v[0].at[pl.ds(0, E_TAIL)]
        tv = ef_v[0].at[pl.ds(0, E_TAIL)]
        pltpu.sync_copy(dst_hbm.at[pl.ds(base, E_TAIL)], ti)
        pltpu.sync_copy(ef_hbm.at[pl.ds(base, E_TAIL)], tv)
        pltpu.sync_copy(tv, accum_sh.at[ti], add=True)
    plsc.subcore_barrier()
    pltpu.sync_copy(accum_sh.at[pl.ds(row0, ROWS1D_PER_TILE)],
                    parts_hbm.at[c, pl.ds(row0, ROWS1D_PER_TILE)])


BLK = 1000


def _layer_body(h_ref, p_ref, heb_ref, wh_ref, wn_ref, we_ref, b_ref,
                out_ref, *, act):
    z = jnp.dot(h_ref[...], wh_ref[...], preferred_element_type=jnp.float32)
    z = z + jnp.dot(p_ref[0] + p_ref[1], wn_ref[...],
                    preferred_element_type=jnp.float32)
    z = z + heb_ref[...] * we_ref[...]
    z = z + b_ref[...]
    if act == "relu_res":
        out_ref[...] = jnp.maximum(z, 0.0) + z
    elif act == "sigmoid":
        out_ref[...] = jax.nn.sigmoid(z)
    else:
        out_ref[...] = z


def _make_layer(act):
    return pl.pallas_call(
        functools.partial(_layer_body, act=act),
        grid=(N // BLK,),
        in_specs=[
            pl.BlockSpec((BLK, D), lambda i: (i, 0)),
            pl.BlockSpec((NC, BLK, D), lambda i: (0, i, 0)),
            pl.BlockSpec((BLK, D), lambda i: (i, 0)),
            pl.BlockSpec((D, D), lambda i: (0, 0)),
            pl.BlockSpec((D, D), lambda i: (0, 0)),
            pl.BlockSpec((1, D), lambda i: (0, 0)),
            pl.BlockSpec((1, D), lambda i: (0, 0)),
        ],
        out_specs=pl.BlockSpec((BLK, D), lambda i: (i, 0)),
        out_shape=jax.ShapeDtypeStruct((N, D), jnp.float32),
    )


_layer_relu = _make_layer("relu_res")
_layer_sigmoid = _make_layer("sigmoid")
_layer_none = _make_layer("none")


def _he_prep_body(parts_ref, out_ref):
    colsum = jnp.sum(parts_ref[...], axis=1, keepdims=True)
    out_ref[...] = jnp.broadcast_to(colsum, (BLK, D))


_he_prep = pl.pallas_call(
    _he_prep_body,
    grid=(N // BLK,),
    in_specs=[pl.BlockSpec((BLK, NC), lambda i: (i, 0))],
    out_specs=pl.BlockSpec((BLK, D), lambda i: (i, 0)),
    out_shape=jax.ShapeDtypeStruct((N, D), jnp.float32),
)


def kernel(node_feat, edge_feat, edge_index, Ws, bs):
    src = edge_index[0]
    dst = edge_index[1]
    ef = edge_feat[:, 0]
    zeros_tile = jnp.zeros((ZROWS, D), jnp.float32)
    zeros_1d = jnp.zeros((ROWS1D_PER_TILE,), jnp.float32)

    he_parts = _seg_sum_edge_scalar(dst, ef, zeros_1d)
    he_b = _he_prep(he_parts.T)

    Wh = Ws[:, :D, :]
    Wn = Ws[:, D:2 * D, :]
    We = Ws[:, 2 * D:2 * D + 1, :]
    Bs = bs[:, None, :]

    acts = ["relu_res"] * 5 + ["sigmoid"] + ["relu_res", "relu_res",
                                             "sigmoid", "none"]
    h = node_feat
    for i, act in enumerate(acts):
        parts = _seg_sum_rows(h, src, dst, zeros_tile)
        layer = (_layer_relu if act == "relu_res"
                 else _layer_sigmoid if act == "sigmoid" else _layer_none)
        h = layer(h, parts, he_b, Wh[i], Wn[i], We[i], Bs[i])
    return h


# speedup vs baseline: 1.1404x; 1.1404x over previous
import functools

import jax
import jax.numpy as jnp
from jax import lax
from jax.experimental import pallas as pl
from jax.experimental.pallas import tpu as pltpu
from jax.experimental.pallas import tpu_sc as plsc

N = 10000
E = 320000
D = 128
NC = 2
NS = 16
NW = NC * NS
E_PER_SC = E // NC
E_PER_TILE = E_PER_SC // NS
CH = 80
N_CHUNKS = E_PER_TILE // CH
E_TAIL = E_PER_TILE - N_CHUNKS * CH
ROWS_PER_TILE = 632
N_ROW_PAD = NS * ROWS_PER_TILE
N_PAD = 10240
ROWS1D_PER_TILE = N_PAD // NS

_mesh = plsc.VectorSubcoreMesh(core_axis_name="c", subcore_axis_name="s")


NBUF = 3
N_MAIN = (N_CHUNKS // NBUF) * NBUF
N_GROUPS = N_MAIN // NBUF
ZCOPY_TILES = 10
ZROWS = N // ZCOPY_TILES


@functools.partial(
    pl.kernel,
    mesh=_mesh,
    out_type=jax.ShapeDtypeStruct((NC, N, D), jnp.float32),
    scratch_types=[
        pltpu.VMEM((E_PER_TILE,), jnp.int32),
        pltpu.VMEM((E_PER_TILE,), jnp.int32),
        [pltpu.VMEM((CH, D), jnp.float32) for _ in range(NBUF)],
        [pltpu.SemaphoreType.DMA for _ in range(NBUF)],
        [pltpu.SemaphoreType.DMA for _ in range(NBUF)],
        pltpu.SemaphoreType.DMA,
        pltpu.SemaphoreType.DMA,
        pltpu.VMEM_SHARED((N, D), jnp.float32),
    ],
)
def _seg_sum_rows(h_hbm, src_hbm, dst_hbm, zeros_hbm, parts_hbm,
                  src_v, dst_v, rows_v, gsems, ssems, isem0, isem1, accum_sh):
    c = lax.axis_index("c")
    s = lax.axis_index("s")
    ebase = c * E_PER_SC + s * E_PER_TILE
    pltpu.async_copy(src_hbm.at[pl.ds(ebase, E_PER_TILE)], src_v, isem0)
    pltpu.async_copy(dst_hbm.at[pl.ds(ebase, E_PER_TILE)], dst_v, isem1)

    @pl.when(s < ZCOPY_TILES)
    def _():
        pltpu.sync_copy(zeros_hbm, accum_sh.at[pl.ds(s * ZROWS, ZROWS)])

    pltpu.make_async_copy(src_hbm.at[pl.ds(ebase, E_PER_TILE)], src_v,
                          isem0).wait()
    pltpu.make_async_copy(dst_hbm.at[pl.ds(ebase, E_PER_TILE)], dst_v,
                          isem1).wait()
    plsc.subcore_barrier()

    def sidx(chunk):
        return src_v.at[pl.ds(chunk * CH, CH)]

    def didx(chunk):
        return dst_v.at[pl.ds(chunk * CH, CH)]

    def fire(chunk, b):
        pltpu.async_copy(h_hbm.at[sidx(chunk)], rows_v[b], gsems[b])

    def wait_gather_start_scatter(chunk, b):
        pltpu.make_async_copy(h_hbm.at[sidx(chunk)], rows_v[b],
                              gsems[b]).wait()
        pltpu.async_copy(rows_v[b], accum_sh.at[didx(chunk)], ssems[b],
                         add=True)

    def wait_scatter(chunk, b):
        pltpu.make_async_copy(rows_v[b], accum_sh.at[didx(chunk)],
                              ssems[b]).wait()

    for b in range(NBUF):
        fire(b, b)

    def body(g, carry):
        for b in range(NBUF):
            wait_gather_start_scatter(g * NBUF + b, b)
        for b in range(NBUF):
            wait_scatter(g * NBUF + b, b)
            fire((g + 1) * NBUF + b, b)
        return carry

    lax.fori_loop(0, N_GROUPS - 1, body, 0)
    g_last = N_GROUPS - 1
    for b in range(NBUF):
        wait_gather_start_scatter(g_last * NBUF + b, b)
    for b in range(NBUF):
        wait_scatter(g_last * NBUF + b, b)
    for t in range(N_MAIN, N_CHUNKS):
        fire(t, 0)
        wait_gather_start_scatter(t, 0)
        wait_scatter(t, 0)

    if E_TAIL:
        ti = src_v.at[pl.ds(N_CHUNKS * CH, E_TAIL)]
        to = dst_v.at[pl.ds(N_CHUNKS * CH, E_TAIL)]
        tr = rows_v[0].at[pl.ds(0, E_TAIL)]
        pltpu.async_copy(h_hbm.at[ti], tr, gsems[0])
        pltpu.make_async_copy(h_hbm.at[ti], tr, gsems[0]).wait()
        pltpu.sync_copy(tr, accum_sh.at[to], add=True)

    plsc.subcore_barrier()

    @pl.when(s < ZCOPY_TILES)
    def _():
        pltpu.sync_copy(accum_sh.at[pl.ds(s * ZROWS, ZROWS)],
                        parts_hbm.at[c, pl.ds(s * ZROWS, ZROWS)])


@functools.partial(
    pl.kernel,
    mesh=_mesh,
    out_type=jax.ShapeDtypeStruct((NC, N_PAD), jnp.float32),
    scratch_types=[
        [pltpu.VMEM((CH,), jnp.int32) for _ in range(NBUF)],
        [pltpu.VMEM((CH,), jnp.float32) for _ in range(NBUF)],
        [pltpu.SemaphoreType.DMA for _ in range(NBUF)],
        pltpu.VMEM_SHARED((N_PAD,), jnp.float32),
    ],
)
def _seg_sum_edge_scalar(dst_hbm, ef_hbm, zeros1d_hbm, parts_hbm,
                         dst_v, ef_v, sems, accum_sh):
    c = lax.axis_index("c")
    s = lax.axis_index("s")
    row0 = s * ROWS1D_PER_TILE
    pltpu.sync_copy(zeros1d_hbm, accum_sh.at[pl.ds(row0, ROWS1D_PER_TILE)])
    plsc.subcore_barrier()

    ebase = c * E_PER_SC + s * E_PER_TILE

    def body(i, carry):
        for b in range(NBUF):
            base = ebase + (i * NBUF + b) * CH
            pltpu.sync_copy(dst_hbm.at[pl.ds(base, CH)], dst_v[b])
            pltpu.sync_copy(ef_hbm.at[pl.ds(base, CH)], ef_v[b])
            pltpu.async_copy(ef_v[b], accum_sh.at[dst_v[b]], sems[b],
                             add=True)
        for b in range(NBUF):
            pltpu.make_async_copy(ef_v[b], accum_sh.at[dst_v[b]],
                                  sems[b]).wait()
        return carry

    lax.fori_loop(0, N_CHUNKS // NBUF, body, 0)
    for t in range(N_MAIN, N_CHUNKS):
        base = ebase + t * CH
        pltpu.sync_copy(dst_hbm.at[pl.ds(base, CH)], dst_v[0])
        pltpu.sync_copy(ef_hbm.at[pl.ds(base, CH)], ef_v[0])
        pltpu.sync_copy(ef_v[0], accum_sh.at[dst_v[0]], add=True)
    if E_TAIL:
        base = ebase + N_CHUNKS * CH
        ti = dst_v[0].at[pl.ds(0, E_TAIL)]
        tv = ef_v[0].at[pl.ds(0, E_TAIL)]
        pltpu.sync_copy(dst_hbm.at[pl.ds(base, E_TAIL)], ti)
        pltpu.sync_copy(ef_hbm.at[pl.ds(base, E_TAIL)], tv)
        pltpu.sync_copy(tv, accum_sh.at[ti], add=True)
    plsc.subcore_barrier()
    pltpu.sync_copy(accum_sh.at[pl.ds(row0, ROWS1D_PER_TILE)],
                    parts_hbm.at[c, pl.ds(row0, ROWS1D_PER_TILE)])


BLK = 1000


def _layer_body(h_ref, p_ref, heb_ref, wh_ref, wn_ref, we_ref, b_ref,
                out_ref, *, act):
    z = jnp.dot(h_ref[...], wh_ref[...], preferred_element_type=jnp.float32)
    z = z + jnp.dot(p_ref[0] + p_ref[1], wn_ref[...],
                    preferred_element_type=jnp.float32)
    z = z + heb_ref[...] * we_ref[...]
    z = z + b_ref[...]
    if act == "relu_res":
        out_ref[...] = jnp.maximum(z, 0.0) + z
    elif act == "sigmoid":
        out_ref[...] = jax.nn.sigmoid(z)
    else:
        out_ref[...] = z


def _make_layer(act):
    return pl.pallas_call(
        functools.partial(_layer_body, act=act),
        grid=(N // BLK,),
        in_specs=[
            pl.BlockSpec((BLK, D), lambda i: (i, 0)),
            pl.BlockSpec((NC, BLK, D), lambda i: (0, i, 0)),
            pl.BlockSpec((BLK, D), lambda i: (i, 0)),
            pl.BlockSpec((D, D), lambda i: (0, 0)),
            pl.BlockSpec((D, D), lambda i: (0, 0)),
            pl.BlockSpec((1, D), lambda i: (0, 0)),
            pl.BlockSpec((1, D), lambda i: (0, 0)),
        ],
        out_specs=pl.BlockSpec((BLK, D), lambda i: (i, 0)),
        out_shape=jax.ShapeDtypeStruct((N, D), jnp.float32),
    )


_layer_relu = _make_layer("relu_res")
_layer_sigmoid = _make_layer("sigmoid")
_layer_none = _make_layer("none")


def _he_prep_body(parts_ref, out_ref):
    colsum = jnp.sum(parts_ref[...], axis=1, keepdims=True)
    out_ref[...] = jnp.broadcast_to(colsum, (BLK, D))


_he_prep = pl.pallas_call(
    _he_prep_body,
    grid=(N // BLK,),
    in_specs=[pl.BlockSpec((BLK, NC), lambda i: (i, 0))],
    out_specs=pl.BlockSpec((BLK, D), lambda i: (i, 0)),
    out_shape=jax.ShapeDtypeStruct((N, D), jnp.float32),
)


def kernel(node_feat, edge_feat, edge_index, Ws, bs):
    src = edge_index[0]
    dst = edge_index[1]
    ef = edge_feat[:, 0]
    zeros_tile = jnp.zeros((ZROWS, D), jnp.float32)
    zeros_1d = jnp.zeros((ROWS1D_PER_TILE,), jnp.float32)

    he_parts = _seg_sum_edge_scalar(dst, ef, zeros_1d)
    he_b = _he_prep(he_parts.T)

    Wh = Ws[:, :D, :]
    Wn = Ws[:, D:2 * D, :]
    We = Ws[:, 2 * D:2 * D + 1, :]
    Bs = bs[:, None, :]

    acts = ["relu_res"] * 5 + ["sigmoid"] + ["relu_res", "relu_res",
                                             "sigmoid", "none"]
    h = node_feat
    for i, act in enumerate(acts):
        parts = _seg_sum_rows(h, src, dst, zeros_tile)
        layer = (_layer_relu if act == "relu_res"
                 else _layer_sigmoid if act == "sigmoid" else _layer_none)
        h = layer(h, parts, he_b, Wh[i], Wn[i], We[i], Bs[i])
    return h
